# baseline (speedup 1.0000x reference)
import functools

import jax
import jax.numpy as jnp
from jax import lax
from jax.experimental import pallas as pl
from jax.experimental.pallas import tpu as pltpu
from jax.experimental.pallas import tpu_sc as plsc

_NC = 2
_NS = 16
_NW = _NC * _NS
_CHUNK = 128


@functools.partial(jax.jit, static_argnames=("n_rows", "d", "seq_len"))
def _sc_embed(table, idx2d, pos, *, n_rows, d, seq_len):
    b_per_w = n_rows // _NW
    chunks_per_w = b_per_w // _CHUNK
    reps = b_per_w // seq_len

    mesh = plsc.VectorSubcoreMesh(
        core_axis_name="c", subcore_axis_name="s",
        num_cores=_NC, num_subcores=_NS)

    @functools.partial(
        pl.kernel,
        out_type=jax.ShapeDtypeStruct((n_rows, d), jnp.float32),
        mesh=mesh,
        scratch_types=[
            pltpu.VMEM((chunks_per_w, _CHUNK), jnp.int32),
            pltpu.VMEM((b_per_w, d), jnp.float32),
            pltpu.SemaphoreType.DMA,
        ],
        compiler_params=pltpu.CompilerParams(use_tc_tiling_on_sc=False),
    )
    def k(table_hbm, idx_hbm, pos_hbm, out_hbm, idx_v, rows_v, sem):
        wid = lax.axis_index("s") * _NC + lax.axis_index("c")
        base = wid * b_per_w
        pltpu.sync_copy(idx_hbm.at[pl.ds(wid * chunks_per_w, chunks_per_w), :],
                        idx_v)
        for r in range(reps):
            pltpu.sync_copy(pos_hbm, rows_v.at[pl.ds(r * seq_len, seq_len), :])
        copies = []
        for j in range(chunks_per_w):
            copies.append(pltpu.async_copy(
                table_hbm.at[idx_v.at[j]],
                rows_v.at[pl.ds(j * _CHUNK, _CHUNK), :],
                sem, add=True))
        for c in copies:
            c.wait()
        pltpu.sync_copy(rows_v, out_hbm.at[pl.ds(base, b_per_w), :])

    return k(table, idx2d, pos)


def kernel(x, token_table, pos_embedding):
    B, L = x.shape
    d = token_table.shape[1]
    n_rows = B * L
    idx2d = x.reshape(n_rows // _CHUNK, _CHUNK).astype(jnp.int32)
    pos = pos_embedding[0, :L, :]
    out = _sc_embed(token_table, idx2d, pos, n_rows=n_rows, d=d, seq_len=L)
    out = out.reshape(B, L, d)
    real, imag = jnp.split(out, 2, axis=-1)
    return jax.lax.complex(real, imag)


# speedup vs baseline: 1.5871x; 1.5871x over previous
import functools

import jax
import jax.numpy as jnp
from jax import lax
from jax.experimental import pallas as pl
from jax.experimental.layout import Format, Layout, with_layout_constraint
from jax.experimental.pallas import tpu as pltpu
from jax.experimental.pallas import tpu_sc as plsc

_NC = 2
_NS = 16
_NW = _NC * _NS
_CHUNK = 128


@functools.partial(jax.jit, static_argnames=("n_rows", "d", "seq_len"))
def _sc_embed(table, idx2d, pos, *, n_rows, d, seq_len):
    b_per_w = n_rows // _NW
    chunks_per_w = b_per_w // _CHUNK
    reps = b_per_w // seq_len
    h = d // 2

    mesh = plsc.VectorSubcoreMesh(
        core_axis_name="c", subcore_axis_name="s",
        num_cores=_NC, num_subcores=_NS)

    @functools.partial(
        pl.kernel,
        out_type=(jax.ShapeDtypeStruct((n_rows, h), jnp.float32),
                  jax.ShapeDtypeStruct((n_rows, h), jnp.float32)),
        mesh=mesh,
        scratch_types=[
            pltpu.VMEM((chunks_per_w, _CHUNK), jnp.int32),
            pltpu.VMEM((b_per_w, d), jnp.float32),
            pltpu.SemaphoreType.DMA,
        ],
        compiler_params=pltpu.CompilerParams(use_tc_tiling_on_sc=False),
    )
    def k(table_hbm, idx_hbm, pos_hbm, re_hbm, im_hbm, idx_v, rows_v, sem):
        wid = lax.axis_index("s") * _NC + lax.axis_index("c")
        base = wid * b_per_w
        pltpu.sync_copy(idx_hbm.at[pl.ds(wid * chunks_per_w, chunks_per_w), :],
                        idx_v)
        for r in range(reps):
            pltpu.sync_copy(pos_hbm, rows_v.at[pl.ds(r * seq_len, seq_len), :])
        copies = []
        for j in range(chunks_per_w):
            copies.append(pltpu.async_copy(
                table_hbm.at[idx_v.at[j]],
                rows_v.at[pl.ds(j * _CHUNK, _CHUNK), :],
                sem, add=True))
        for c in copies:
            c.wait()
        pltpu.sync_copy(rows_v.at[:, pl.ds(0, h)],
                        re_hbm.at[pl.ds(base, b_per_w), :])
        pltpu.sync_copy(rows_v.at[:, pl.ds(h, h)],
                        im_hbm.at[pl.ds(base, b_per_w), :])

    return k(table, idx2d, pos)


def kernel(x, token_table, pos_embedding):
    B, L = x.shape
    d = token_table.shape[1]
    n_rows = B * L
    idx2d = x.reshape(n_rows // _CHUNK, _CHUNK).astype(jnp.int32)
    pos = pos_embedding[0, :L, :]
    table_lin = with_layout_constraint(
        token_table,
        Layout(major_to_minor=(0, 1), tiling=((8, d),)))
    re, im = _sc_embed(table_lin, idx2d, pos, n_rows=n_rows, d=d, seq_len=L)
    re = re.reshape(B, L, d // 2)
    im = im.reshape(B, L, d // 2)
    return jax.lax.complex(re, im)
